# baseline (speedup 1.0000x reference)
import jax
import jax.numpy as jnp
from jax import lax
from jax.experimental import pallas as pl
from jax.experimental.pallas import tpu as pltpu
from jax.experimental.pallas import tpu_sc as plsc

N = 10000
NPAD = 10240
H = 128
CH = 128
NW = 32
NS = 16

_mesh = plsc.VectorSubcoreMesh(core_axis_name="c", subcore_axis_name="s")


def _pad_chunks(idx, mult, pad_row):
    pad = (-idx.shape[0]) % mult
    if pad:
        idx = jnp.concatenate([idx, jnp.broadcast_to(pad_row, (pad, CH))])
    return idx


def _sc_gather(table, idxa, idxb):
    nloc = idxa.shape[0] // NW

    def body(table_hbm, idxa_hbm, idxb_hbm, out_hbm,
             idxva, idxvb, bufa, bufb, sa0, sa1, sb0, sb1):
        w = lax.axis_index("s") * 2 + lax.axis_index("c")
        pltpu.sync_copy(idxa_hbm.at[pl.ds(w * nloc, nloc)], idxva)
        pltpu.sync_copy(idxb_hbm.at[pl.ds(w * nloc, nloc)], idxvb)
        sems_a = (sa0, sa1)
        sems_b = (sb0, sb1)
        pend = [None, None]
        for j in range(nloc + 1):
            p = j & 1
            if j < nloc:
                da = pltpu.async_copy(table_hbm.at[idxva.at[j]],
                                      bufa.at[p], sems_a[p])
                db = pltpu.async_copy(table_hbm.at[idxvb.at[j]],
                                      bufb.at[p], sems_b[p])
            if j >= 1:
                q = (j - 1) & 1
                chunk = w * nloc + (j - 1)
                pa, pb = pend[q]
                pa.wait()
                pltpu.sync_copy(bufa.at[q],
                                out_hbm.at[pl.ds(chunk * CH, CH), pl.ds(0, H)])
                pb.wait()
                pltpu.sync_copy(bufb.at[q],
                                out_hbm.at[pl.ds(chunk * CH, CH), pl.ds(H, H)])
            if j < nloc:
                pend[p] = (da, db)

    f = pl.kernel(
        body,
        out_type=jax.ShapeDtypeStruct((idxa.shape[0] * CH, 2 * H),
                                      jnp.float32),
        mesh=_mesh,
        scratch_types=[pltpu.VMEM((nloc, CH), jnp.int32),
                       pltpu.VMEM((nloc, CH), jnp.int32),
                       pltpu.VMEM((2, CH, H), jnp.float32),
                       pltpu.VMEM((2, CH, H), jnp.float32),
                       pltpu.SemaphoreType.DMA,
                       pltpu.SemaphoreType.DMA,
                       pltpu.SemaphoreType.DMA,
                       pltpu.SemaphoreType.DMA],
    )
    return f(table, idxa, idxb)


def _sc_scatter(rows0, idxa, idxb, idx1, zeros_hbm, zc_hbm):
    n0 = idxa.shape[0] // NW
    n1 = idx1.shape[0] // NW
    slab = NPAD // NS

    def body(rows0_hbm, idxa_hbm, idxb_hbm, idx1_hbm, z_hbm, zc, out_hbm,
             cnt_hbm, ga, gb, idxv1, buf, onesv, acc, acc_cnt, s0, s1):
        c = lax.axis_index("c")
        s = lax.axis_index("s")
        w = s * 2 + c
        pltpu.sync_copy(z_hbm.at[pl.ds(s * slab, slab)],
                        acc.at[pl.ds(s * slab, slab)])
        pltpu.sync_copy(zc.at[pl.ds(s * slab, slab)],
                        acc_cnt.at[pl.ds(s * slab, slab)])
        ones = jnp.ones((16,), jnp.int32)
        for k in range(CH // 16):
            onesv[pl.ds(k * 16, 16)] = ones
        plsc.subcore_barrier()

        pltpu.sync_copy(idx1_hbm.at[pl.ds(w * n1, n1)], idxv1)

        sems = (s0, s1)
        T = 2 * n0
        pend = [None, None]
        for t in range(T + 1):
            p = t & 1
            if t < T:
                j, st = t >> 1, t & 1
                if st == 0 and j % 8 == 0:
                    g = j // 8
                    gp = g & 1
                    pltpu.sync_copy(
                        idxa_hbm.at[pl.ds(w * n0 + g * 8, 8)], ga.at[gp])
                    pltpu.sync_copy(
                        idxb_hbm.at[pl.ds(w * n0 + g * 8, 8)], gb.at[gp])
                chunk = w * n0 + j
                cols = pl.ds(0, H) if st == 0 else pl.ds(H, H)
                r = pltpu.async_copy(rows0_hbm.at[pl.ds(chunk * CH, CH), cols],
                                     buf.at[p], sems[p])
            if t >= 1:
                q = (t - 1) & 1
                j1, st1 = (t - 1) >> 1, (t - 1) & 1
                gp1 = (j1 // 8) & 1
                idxref = (ga if st1 == 0 else gb).at[gp1, j1 % 8]
                pend[q].wait()
                pltpu.sync_copy(buf.at[q], acc.at[idxref], add=True)
            if t < T:
                pend[p] = r

        def it1(j, carry):
            pltpu.sync_copy(onesv, acc_cnt.at[idxv1.at[j]], add=True)
            return carry

        lax.fori_loop(0, n1, it1, 0)

        plsc.subcore_barrier()
        pltpu.sync_copy(acc.at[pl.ds(s * slab, slab)],
                        out_hbm.at[c, pl.ds(s * slab, slab)])
        pltpu.sync_copy(acc_cnt.at[pl.ds(s * slab, slab)],
                        cnt_hbm.at[c, pl.ds(s * slab, slab)])

    f = pl.kernel(
        body,
        out_type=(jax.ShapeDtypeStruct((2, NPAD, H), jnp.float32),
                  jax.ShapeDtypeStruct((2, NPAD), jnp.int32)),
        mesh=_mesh,
        scratch_types=[pltpu.VMEM((2, 8, CH), jnp.int32),
                       pltpu.VMEM((2, 8, CH), jnp.int32),
                       pltpu.VMEM((n1, CH), jnp.int32),
                       pltpu.VMEM((2, CH, H), jnp.float32),
                       pltpu.VMEM((CH,), jnp.int32),
                       pltpu.VMEM_SHARED((NPAD, H), jnp.float32),
                       pltpu.VMEM_SHARED((NPAD,), jnp.int32),
                       pltpu.SemaphoreType.DMA,
                       pltpu.SemaphoreType.DMA],
    )
    return f(rows0, idxa, idxb, idx1, zeros_hbm, zc_hbm)


def _mlp_body(x_ref, wa_ref, ba_ref, wb_ref, bb_ref, o_ref):
    x = x_ref[...].astype(jnp.bfloat16)
    wa = wa_ref[...].astype(jnp.bfloat16)
    h = lax.dot_general(x, wa, (((1,), (1,)), ((), ())),
                        preferred_element_type=jnp.float32)
    h = jnp.maximum(h + ba_ref[...], 0.0).astype(jnp.bfloat16)
    wb = wb_ref[...].astype(jnp.bfloat16)
    o = lax.dot_general(h, wb, (((1,), (1,)), ((), ())),
                        preferred_element_type=jnp.float32)
    o_ref[...] = o + bb_ref[...]


def _tc_mlp(x, wa, ba, wb, bb, bm):
    m, k = x.shape
    ko = wb.shape[0]
    return pl.pallas_call(
        _mlp_body,
        grid=(m // bm,),
        in_specs=[
            pl.BlockSpec((bm, k), lambda i: (i, 0)),
            pl.BlockSpec(wa.shape, lambda i: (0, 0)),
            pl.BlockSpec((1, ba.shape[0]), lambda i: (0, 0)),
            pl.BlockSpec(wb.shape, lambda i: (0, 0)),
            pl.BlockSpec((1, bb.shape[0]), lambda i: (0, 0)),
        ],
        out_specs=pl.BlockSpec((bm, ko), lambda i: (i, 0)),
        out_shape=jax.ShapeDtypeStruct((m, ko), jnp.float32),
    )(x, wa, ba.reshape(1, -1), wb, bb.reshape(1, -1))


def _update_body(p0_ref, p1_ref, cnt_ref, ns_ref, w1a_ref, b1a_ref, w1b_ref,
                 b1b_ref, wu1_ref, bu1_ref, wu2_ref, bu2_ref, o_ref):
    ns = ns_ref[...]
    h1 = lax.dot_general(ns, w1a_ref[...], (((1,), (1,)), ((), ())),
                         preferred_element_type=jnp.float32)
    h1 = jnp.maximum(h1 + b1a_ref[...], 0.0)
    m1 = lax.dot_general(h1, w1b_ref[...], (((1,), (1,)), ((), ())),
                         preferred_element_type=jnp.float32)
    m1 = m1 + b1b_ref[...]
    sm = p0_ref[...] + p1_ref[...] + cnt_ref[...] * m1
    wu1 = wu1_ref[...]
    h = lax.dot_general(sm, wu1[:, :H], (((1,), (1,)), ((), ())),
                        preferred_element_type=jnp.float32)
    h = h + lax.dot_general(ns, wu1[:, H:], (((1,), (1,)), ((), ())),
                            preferred_element_type=jnp.float32)
    h = jnp.maximum(h + bu1_ref[...], 0.0)
    o = lax.dot_general(h, wu2_ref[...], (((1,), (1,)), ((), ())),
                        preferred_element_type=jnp.float32)
    o_ref[...] = o + bu2_ref[...]


def _tc_update(p0, p1, cnt, ns, w1a, b1a, w1b, b1b, wu1, bu1, wu2, bu2):
    bm = 1280
    row = lambda i: (i, 0)
    fix = lambda i: (0, 0)
    return pl.pallas_call(
        _update_body,
        grid=(NPAD // bm,),
        in_specs=[
            pl.BlockSpec((bm, H), row),
            pl.BlockSpec((bm, H), row),
            pl.BlockSpec((bm, 1), row),
            pl.BlockSpec((bm, H), row),
            pl.BlockSpec(w1a.shape, fix),
            pl.BlockSpec((1, H), fix),
            pl.BlockSpec(w1b.shape, fix),
            pl.BlockSpec((1, H), fix),
            pl.BlockSpec(wu1.shape, fix),
            pl.BlockSpec((1, 2 * H), fix),
            pl.BlockSpec(wu2.shape, fix),
            pl.BlockSpec((1, H), fix),
        ],
        out_specs=pl.BlockSpec((bm, H), row),
        out_shape=jax.ShapeDtypeStruct((NPAD, H), jnp.float32),
    )(p0, p1, cnt, ns, w1a, b1a.reshape(1, -1), w1b, b1b.reshape(1, -1),
      wu1, bu1.reshape(1, -1), wu2, bu2.reshape(1, -1))


def kernel(node_states, rel0, rel1, W0a, b0a, W0b, b0b, W1a, b1a, W1b, b1b,
           Wu1, bu1, Wu2, bu2):
    r0 = rel0.astype(jnp.int32)
    idxa = r0[0::2].reshape(-1, CH)
    idxb = r0[1::2].reshape(-1, CH)
    idx1 = rel1.astype(jnp.int32).reshape(-1, CH)
    lane = jnp.arange(CH, dtype=jnp.int32)
    spad = N + lane
    idxa_p = _pad_chunks(idxa, NW * 8, spad)
    idxb_p = _pad_chunks(idxb, NW * 8, spad)
    idx1_p = _pad_chunks(idx1, NW * 8, spad)
    table = jnp.concatenate(
        [node_states, jnp.zeros((NPAD - N, H), jnp.float32)])

    inp0 = _sc_gather(table, idxa_p, idxb_p)
    out0 = _tc_mlp(inp0, W0a, b0a, W0b, b0b, bm=4096)

    zeros = jnp.zeros((NPAD, H), jnp.float32)
    zcnt = jnp.zeros((NPAD,), jnp.int32)
    partials, cnts = _sc_scatter(out0, idxa_p, idxb_p, idx1_p, zeros, zcnt)

    cnt = (cnts[0] + cnts[1]).astype(jnp.float32).reshape(NPAD, 1)
    out = _tc_update(partials[0], partials[1], cnt, table,
                     W1a, b1a, W1b, b1b, Wu1, bu1, Wu2, bu2)
    return out[:N]


# speedup vs baseline: 1.0290x; 1.0290x over previous
import jax
import jax.numpy as jnp
from jax import lax
from jax.experimental import pallas as pl
from jax.experimental.pallas import tpu as pltpu
from jax.experimental.pallas import tpu_sc as plsc

N = 10000
NPAD = 10240
H = 128
CH = 128
NW = 32
NS = 16

_mesh = plsc.VectorSubcoreMesh(core_axis_name="c", subcore_axis_name="s")


def _pad_chunks(idx, mult, pad_row):
    pad = (-idx.shape[0]) % mult
    if pad:
        idx = jnp.concatenate([idx, jnp.broadcast_to(pad_row, (pad, CH))])
    return idx


def _sc_gather(table, idxa, idxb):
    nloc = idxa.shape[0] // NW

    def body(table_hbm, idxa_hbm, idxb_hbm, out_hbm,
             idxva, idxvb, bufa, bufb, sa0, sa1, sb0, sb1):
        w = lax.axis_index("s") * 2 + lax.axis_index("c")
        pltpu.sync_copy(idxa_hbm.at[pl.ds(w * nloc, nloc)], idxva)
        pltpu.sync_copy(idxb_hbm.at[pl.ds(w * nloc, nloc)], idxvb)
        sems_a = (sa0, sa1)
        sems_b = (sb0, sb1)
        pend = [None, None]
        for j in range(nloc + 1):
            p = j & 1
            if j < nloc:
                da = pltpu.async_copy(table_hbm.at[idxva.at[j]],
                                      bufa.at[p], sems_a[p])
                db = pltpu.async_copy(table_hbm.at[idxvb.at[j]],
                                      bufb.at[p], sems_b[p])
            if j >= 1:
                q = (j - 1) & 1
                chunk = w * nloc + (j - 1)
                pa, pb = pend[q]
                pa.wait()
                pltpu.sync_copy(bufa.at[q],
                                out_hbm.at[pl.ds(chunk * CH, CH), pl.ds(0, H)])
                pb.wait()
                pltpu.sync_copy(bufb.at[q],
                                out_hbm.at[pl.ds(chunk * CH, CH), pl.ds(H, H)])
            if j < nloc:
                pend[p] = (da, db)

    f = pl.kernel(
        body,
        out_type=jax.ShapeDtypeStruct((idxa.shape[0] * CH, 2 * H),
                                      jnp.float32),
        mesh=_mesh,
        scratch_types=[pltpu.VMEM((nloc, CH), jnp.int32),
                       pltpu.VMEM((nloc, CH), jnp.int32),
                       pltpu.VMEM((2, CH, H), jnp.float32),
                       pltpu.VMEM((2, CH, H), jnp.float32),
                       pltpu.SemaphoreType.DMA,
                       pltpu.SemaphoreType.DMA,
                       pltpu.SemaphoreType.DMA,
                       pltpu.SemaphoreType.DMA],
    )
    return f(table, idxa, idxb)


def _sc_scatter(rows0, idxa, idxb, idx1, zeros_hbm, zc_hbm):
    n0 = idxa.shape[0] // NW
    n1 = idx1.shape[0] // NW
    slab = NPAD // NS

    def body(rows0_hbm, idxa_hbm, idxb_hbm, idx1_hbm, z_hbm, zc, out_hbm,
             cnt_hbm, ga, gb, idxv1, buf, onesv, acc, acc_cnt, s0, s1):
        c = lax.axis_index("c")
        s = lax.axis_index("s")
        w = s * 2 + c
        pltpu.sync_copy(z_hbm.at[pl.ds(s * slab, slab)],
                        acc.at[pl.ds(s * slab, slab)])
        pltpu.sync_copy(zc.at[pl.ds(s * slab, slab)],
                        acc_cnt.at[pl.ds(s * slab, slab)])
        ones = jnp.ones((16,), jnp.int32)
        for k in range(CH // 16):
            onesv[pl.ds(k * 16, 16)] = ones
        plsc.subcore_barrier()

        pltpu.sync_copy(idx1_hbm.at[pl.ds(w * n1, n1)], idxv1)

        sems = (s0, s1)
        T = 2 * n0
        pend = [None, None]
        for t in range(T + 1):
            p = t & 1
            if t < T:
                j, st = t >> 1, t & 1
                if st == 0 and j % 8 == 0:
                    g = j // 8
                    gp = g & 1
                    pltpu.sync_copy(
                        idxa_hbm.at[pl.ds(w * n0 + g * 8, 8)], ga.at[gp])
                    pltpu.sync_copy(
                        idxb_hbm.at[pl.ds(w * n0 + g * 8, 8)], gb.at[gp])
                chunk = w * n0 + j
                cols = pl.ds(0, H) if st == 0 else pl.ds(H, H)
                r = pltpu.async_copy(rows0_hbm.at[pl.ds(chunk * CH, CH), cols],
                                     buf.at[p], sems[p])
            if t >= 1:
                q = (t - 1) & 1
                j1, st1 = (t - 1) >> 1, (t - 1) & 1
                gp1 = (j1 // 8) & 1
                idxref = (ga if st1 == 0 else gb).at[gp1, j1 % 8]
                pend[q].wait()
                pltpu.sync_copy(buf.at[q], acc.at[idxref], add=True)
            if t < T:
                pend[p] = r

        def it1(j, carry):
            pltpu.sync_copy(onesv, acc_cnt.at[idxv1.at[j]], add=True)
            return carry

        lax.fori_loop(0, n1, it1, 0)

        plsc.subcore_barrier()
        pltpu.sync_copy(acc.at[pl.ds(s * slab, slab)],
                        out_hbm.at[c, pl.ds(s * slab, slab)])
        pltpu.sync_copy(acc_cnt.at[pl.ds(s * slab, slab)],
                        cnt_hbm.at[c, pl.ds(s * slab, slab)])

    f = pl.kernel(
        body,
        out_type=(jax.ShapeDtypeStruct((2, NPAD, H), jnp.float32),
                  jax.ShapeDtypeStruct((2, NPAD), jnp.int32)),
        mesh=_mesh,
        scratch_types=[pltpu.VMEM((2, 8, CH), jnp.int32),
                       pltpu.VMEM((2, 8, CH), jnp.int32),
                       pltpu.VMEM((n1, CH), jnp.int32),
                       pltpu.VMEM((2, CH, H), jnp.float32),
                       pltpu.VMEM((CH,), jnp.int32),
                       pltpu.VMEM_SHARED((NPAD, H), jnp.float32),
                       pltpu.VMEM_SHARED((NPAD,), jnp.int32),
                       pltpu.SemaphoreType.DMA,
                       pltpu.SemaphoreType.DMA],
    )
    return f(rows0, idxa, idxb, idx1, zeros_hbm, zc_hbm)


def _mlp_body(x_ref, wa_ref, ba_ref, wb_ref, bb_ref, o_ref):
    x = x_ref[...].astype(jnp.bfloat16)
    wa = wa_ref[...].astype(jnp.bfloat16)
    h = lax.dot_general(x, wa, (((1,), (1,)), ((), ())),
                        preferred_element_type=jnp.float32)
    h = jnp.maximum(h + ba_ref[...], 0.0).astype(jnp.bfloat16)
    wb = wb_ref[...].astype(jnp.bfloat16)
    o = lax.dot_general(h, wb, (((1,), (1,)), ((), ())),
                        preferred_element_type=jnp.float32)
    o_ref[...] = o + bb_ref[...]


def _tc_mlp(x, wa, ba, wb, bb, bm):
    m, k = x.shape
    ko = wb.shape[0]
    return pl.pallas_call(
        _mlp_body,
        grid=(m // bm,),
        in_specs=[
            pl.BlockSpec((bm, k), lambda i: (i, 0)),
            pl.BlockSpec(wa.shape, lambda i: (0, 0)),
            pl.BlockSpec((1, ba.shape[0]), lambda i: (0, 0)),
            pl.BlockSpec(wb.shape, lambda i: (0, 0)),
            pl.BlockSpec((1, bb.shape[0]), lambda i: (0, 0)),
        ],
        out_specs=pl.BlockSpec((bm, ko), lambda i: (i, 0)),
        out_shape=jax.ShapeDtypeStruct((m, ko), jnp.float32),
    )(x, wa, ba.reshape(1, -1), wb, bb.reshape(1, -1))


def _update_body(p0_ref, p1_ref, p2_ref, p3_ref, cnt_ref, ns_ref, w1a_ref,
                 b1a_ref, w1b_ref, b1b_ref, wu1_ref, bu1_ref, wu2_ref,
                 bu2_ref, o_ref):
    ns = ns_ref[...]
    h1 = lax.dot_general(ns, w1a_ref[...], (((1,), (1,)), ((), ())),
                         preferred_element_type=jnp.float32)
    h1 = jnp.maximum(h1 + b1a_ref[...], 0.0)
    m1 = lax.dot_general(h1, w1b_ref[...], (((1,), (1,)), ((), ())),
                         preferred_element_type=jnp.float32)
    m1 = m1 + b1b_ref[...]
    sm = ((p0_ref[...] + p1_ref[...]) + (p2_ref[...] + p3_ref[...])
          + cnt_ref[...] * m1)
    wu1 = wu1_ref[...]
    h = lax.dot_general(sm, wu1[:, :H], (((1,), (1,)), ((), ())),
                        preferred_element_type=jnp.float32)
    h = h + lax.dot_general(ns, wu1[:, H:], (((1,), (1,)), ((), ())),
                            preferred_element_type=jnp.float32)
    h = jnp.maximum(h + bu1_ref[...], 0.0)
    o = lax.dot_general(h, wu2_ref[...], (((1,), (1,)), ((), ())),
                        preferred_element_type=jnp.float32)
    o_ref[...] = o + bu2_ref[...]


def _tc_update(p0, p1, p2, p3, cnt, ns, w1a, b1a, w1b, b1b, wu1, bu1, wu2,
               bu2):
    bm = 1280
    row = lambda i: (i, 0)
    fix = lambda i: (0, 0)
    return pl.pallas_call(
        _update_body,
        grid=(NPAD // bm,),
        in_specs=[
            pl.BlockSpec((bm, H), row),
            pl.BlockSpec((bm, H), row),
            pl.BlockSpec((bm, H), row),
            pl.BlockSpec((bm, H), row),
            pl.BlockSpec((bm, 1), row),
            pl.BlockSpec((bm, H), row),
            pl.BlockSpec(w1a.shape, fix),
            pl.BlockSpec((1, H), fix),
            pl.BlockSpec(w1b.shape, fix),
            pl.BlockSpec((1, H), fix),
            pl.BlockSpec(wu1.shape, fix),
            pl.BlockSpec((1, 2 * H), fix),
            pl.BlockSpec(wu2.shape, fix),
            pl.BlockSpec((1, H), fix),
        ],
        out_specs=pl.BlockSpec((bm, H), row),
        out_shape=jax.ShapeDtypeStruct((NPAD, H), jnp.float32),
    )(p0, p1, p2, p3, cnt, ns, w1a, b1a.reshape(1, -1), w1b,
      b1b.reshape(1, -1), wu1, bu1.reshape(1, -1), wu2, bu2.reshape(1, -1))


def kernel(node_states, rel0, rel1, W0a, b0a, W0b, b0b, W1a, b1a, W1b, b1b,
           Wu1, bu1, Wu2, bu2):
    r0 = rel0.astype(jnp.int32)
    idxa = r0[0::2].reshape(-1, CH)
    idxb = r0[1::2].reshape(-1, CH)
    idx1 = rel1.astype(jnp.int32).reshape(-1, CH)
    lane = jnp.arange(CH, dtype=jnp.int32)
    spad = N + lane
    idxa_p = _pad_chunks(idxa, NW * 8, spad)
    idxb_p = _pad_chunks(idxb, NW * 8, spad)
    c1h = idx1.shape[0] // 2
    idx1_h = [_pad_chunks(idx1[:c1h], NW * 8, spad),
              _pad_chunks(idx1[c1h:], NW * 8, spad)]
    table = jnp.concatenate(
        [node_states, jnp.zeros((NPAD - N, H), jnp.float32)])

    hc = idxa_p.shape[0] // 2
    zeros = jnp.zeros((NPAD, H), jnp.float32)
    zcnt = jnp.zeros((NPAD,), jnp.int32)

    halves = []
    for k in range(2):
        ia = idxa_p[k * hc:(k + 1) * hc]
        ib = idxb_p[k * hc:(k + 1) * hc]
        i1 = idx1_h[k]
        inp = _sc_gather(table, ia, ib)
        out = _tc_mlp(inp, W0a, b0a, W0b, b0b, bm=4096)
        halves.append(_sc_scatter(out, ia, ib, i1, zeros, zcnt))

    (pA, cA), (pB, cB) = halves
    cnt = ((cA[0] + cA[1]) + (cB[0] + cB[1])).astype(jnp.float32)
    out = _tc_update(pA[0], pA[1], pB[0], pB[1], cnt.reshape(NPAD, 1), table,
                     W1a, b1a, W1b, b1b, Wu1, bu1, Wu2, bu2)
    return out[:N]


# speedup vs baseline: 1.0448x; 1.0153x over previous
import jax
import jax.numpy as jnp
from jax import lax
from jax.experimental import pallas as pl
from jax.experimental.pallas import tpu as pltpu
from jax.experimental.pallas import tpu_sc as plsc

N = 10000
NPAD = 10240
H = 128
CH = 128
NW = 32
NS = 16

_mesh = plsc.VectorSubcoreMesh(core_axis_name="c", subcore_axis_name="s")


def _pad_chunks(idx, mult, pad_row):
    pad = (-idx.shape[0]) % mult
    if pad:
        idx = jnp.concatenate([idx, jnp.broadcast_to(pad_row, (pad, CH))])
    return idx


def _sc_gather(table, idxa, idxb):
    nloc = idxa.shape[0] // NW

    def body(table_hbm, idxa_hbm, idxb_hbm, out_hbm,
             idxva, idxvb, bufa, bufb, sa0, sa1, sb0, sb1):
        w = lax.axis_index("s") * 2 + lax.axis_index("c")
        pltpu.sync_copy(idxa_hbm.at[pl.ds(w * nloc, nloc)], idxva)
        pltpu.sync_copy(idxb_hbm.at[pl.ds(w * nloc, nloc)], idxvb)
        sems_a = (sa0, sa1)
        sems_b = (sb0, sb1)
        pend = [None, None]
        for j in range(nloc + 1):
            p = j & 1
            if j < nloc:
                da = pltpu.async_copy(table_hbm.at[idxva.at[j]],
                                      bufa.at[p], sems_a[p])
                db = pltpu.async_copy(table_hbm.at[idxvb.at[j]],
                                      bufb.at[p], sems_b[p])
            if j >= 1:
                q = (j - 1) & 1
                chunk = w * nloc + (j - 1)
                pa, pb = pend[q]
                pa.wait()
                pltpu.sync_copy(bufa.at[q],
                                out_hbm.at[pl.ds(chunk * CH, CH), pl.ds(0, H)])
                pb.wait()
                pltpu.sync_copy(bufb.at[q],
                                out_hbm.at[pl.ds(chunk * CH, CH), pl.ds(H, H)])
            if j < nloc:
                pend[p] = (da, db)

    f = pl.kernel(
        body,
        out_type=jax.ShapeDtypeStruct((idxa.shape[0] * CH, 2 * H),
                                      jnp.float32),
        mesh=_mesh,
        scratch_types=[pltpu.VMEM((nloc, CH), jnp.int32),
                       pltpu.VMEM((nloc, CH), jnp.int32),
                       pltpu.VMEM((2, CH, H), jnp.float32),
                       pltpu.VMEM((2, CH, H), jnp.float32),
                       pltpu.SemaphoreType.DMA,
                       pltpu.SemaphoreType.DMA,
                       pltpu.SemaphoreType.DMA,
                       pltpu.SemaphoreType.DMA],
    )
    return f(table, idxa, idxb)


def _sc_scatter(rows0, idxa, idxb, idx1, zeros_hbm, zc_hbm):
    n0 = idxa.shape[0] // NW
    n1 = idx1.shape[0] // NW
    slab = NPAD // NS

    def body(rows0_hbm, idxa_hbm, idxb_hbm, idx1_hbm, z_hbm, zc, out_hbm,
             cnt_hbm, ga, gb, idxv1, buf, onesv, acc, acc_cnt, s0, s1):
        c = lax.axis_index("c")
        s = lax.axis_index("s")
        w = s * 2 + c
        pltpu.sync_copy(z_hbm.at[pl.ds(s * slab, slab)],
                        acc.at[pl.ds(s * slab, slab)])
        pltpu.sync_copy(zc.at[pl.ds(s * slab, slab)],
                        acc_cnt.at[pl.ds(s * slab, slab)])
        ones = jnp.ones((16,), jnp.int32)
        for k in range(CH // 16):
            onesv[pl.ds(k * 16, 16)] = ones
        plsc.subcore_barrier()

        pltpu.sync_copy(idx1_hbm.at[pl.ds(w * n1, n1)], idxv1)

        sems = (s0, s1)
        T = 2 * n0
        pend = [None, None]
        for t in range(T + 1):
            p = t & 1
            if t < T:
                j, st = t >> 1, t & 1
                if st == 0 and j % 8 == 0:
                    g = j // 8
                    gp = g & 1
                    pltpu.sync_copy(
                        idxa_hbm.at[pl.ds(w * n0 + g * 8, 8)], ga.at[gp])
                    pltpu.sync_copy(
                        idxb_hbm.at[pl.ds(w * n0 + g * 8, 8)], gb.at[gp])
                chunk = w * n0 + j
                cols = pl.ds(0, H) if st == 0 else pl.ds(H, H)
                r = pltpu.async_copy(rows0_hbm.at[pl.ds(chunk * CH, CH), cols],
                                     buf.at[p], sems[p])
            if t >= 1:
                q = (t - 1) & 1
                j1, st1 = (t - 1) >> 1, (t - 1) & 1
                gp1 = (j1 // 8) & 1
                idxref = (ga if st1 == 0 else gb).at[gp1, j1 % 8]
                pend[q].wait()
                pltpu.sync_copy(buf.at[q], acc.at[idxref], add=True)
            if t < T:
                pend[p] = r

        def it1(j, carry):
            pltpu.sync_copy(onesv, acc_cnt.at[idxv1.at[j]], add=True)
            return carry

        lax.fori_loop(0, n1, it1, 0)

        plsc.subcore_barrier()
        pltpu.sync_copy(acc.at[pl.ds(s * slab, slab)],
                        out_hbm.at[c, pl.ds(s * slab, slab)])
        pltpu.sync_copy(acc_cnt.at[pl.ds(s * slab, slab)],
                        cnt_hbm.at[c, pl.ds(s * slab, slab)])

    f = pl.kernel(
        body,
        out_type=(jax.ShapeDtypeStruct((2, NPAD, H), jnp.float32),
                  jax.ShapeDtypeStruct((2, NPAD), jnp.int32)),
        mesh=_mesh,
        scratch_types=[pltpu.VMEM((2, 8, CH), jnp.int32),
                       pltpu.VMEM((2, 8, CH), jnp.int32),
                       pltpu.VMEM((n1, CH), jnp.int32),
                       pltpu.VMEM((2, CH, H), jnp.float32),
                       pltpu.VMEM((CH,), jnp.int32),
                       pltpu.VMEM_SHARED((NPAD, H), jnp.float32),
                       pltpu.VMEM_SHARED((NPAD,), jnp.int32),
                       pltpu.SemaphoreType.DMA,
                       pltpu.SemaphoreType.DMA],
    )
    return f(rows0, idxa, idxb, idx1, zeros_hbm, zc_hbm)


def _mlp_body(x_ref, wa_ref, ba_ref, wb_ref, bb_ref, o_ref):
    x = x_ref[...].astype(jnp.bfloat16)
    wa = wa_ref[...].astype(jnp.bfloat16)
    h = lax.dot_general(x, wa, (((1,), (1,)), ((), ())),
                        preferred_element_type=jnp.float32)
    h = jnp.maximum(h + ba_ref[...], 0.0).astype(jnp.bfloat16)
    wb = wb_ref[...].astype(jnp.bfloat16)
    o = lax.dot_general(h, wb, (((1,), (1,)), ((), ())),
                        preferred_element_type=jnp.float32)
    o_ref[...] = o + bb_ref[...]


def _tc_mlp(x, wa, ba, wb, bb, bm):
    m, k = x.shape
    ko = wb.shape[0]
    return pl.pallas_call(
        _mlp_body,
        grid=(m // bm,),
        in_specs=[
            pl.BlockSpec((bm, k), lambda i: (i, 0)),
            pl.BlockSpec(wa.shape, lambda i: (0, 0)),
            pl.BlockSpec((1, ba.shape[0]), lambda i: (0, 0)),
            pl.BlockSpec(wb.shape, lambda i: (0, 0)),
            pl.BlockSpec((1, bb.shape[0]), lambda i: (0, 0)),
        ],
        out_specs=pl.BlockSpec((bm, ko), lambda i: (i, 0)),
        out_shape=jax.ShapeDtypeStruct((m, ko), jnp.float32),
    )(x, wa, ba.reshape(1, -1), wb, bb.reshape(1, -1))


def _update_body(p0_ref, p1_ref, p2_ref, p3_ref, cnt_ref, ns_ref, w1a_ref,
                 b1a_ref, w1b_ref, b1b_ref, wu1_ref, bu1_ref, wu2_ref,
                 bu2_ref, o_ref):
    ns = ns_ref[...]
    h1 = lax.dot_general(ns, w1a_ref[...], (((1,), (1,)), ((), ())),
                         preferred_element_type=jnp.float32)
    h1 = jnp.maximum(h1 + b1a_ref[...], 0.0)
    m1 = lax.dot_general(h1, w1b_ref[...], (((1,), (1,)), ((), ())),
                         preferred_element_type=jnp.float32)
    m1 = m1 + b1b_ref[...]
    sm = ((p0_ref[...] + p1_ref[...]) + (p2_ref[...] + p3_ref[...])
          + cnt_ref[...] * m1)
    wu1 = wu1_ref[...]
    h = lax.dot_general(sm, wu1[:, :H], (((1,), (1,)), ((), ())),
                        preferred_element_type=jnp.float32)
    h = h + lax.dot_general(ns, wu1[:, H:], (((1,), (1,)), ((), ())),
                            preferred_element_type=jnp.float32)
    h = jnp.maximum(h + bu1_ref[...], 0.0)
    o = lax.dot_general(h, wu2_ref[...], (((1,), (1,)), ((), ())),
                        preferred_element_type=jnp.float32)
    o_ref[...] = o + bu2_ref[...]


def _tc_update(p0, p1, p2, p3, cnt, ns, w1a, b1a, w1b, b1b, wu1, bu1, wu2,
               bu2):
    bm = 1280
    row = lambda i: (i, 0)
    fix = lambda i: (0, 0)
    return pl.pallas_call(
        _update_body,
        grid=(NPAD // bm,),
        in_specs=[
            pl.BlockSpec((bm, H), row),
            pl.BlockSpec((bm, H), row),
            pl.BlockSpec((bm, H), row),
            pl.BlockSpec((bm, H), row),
            pl.BlockSpec((bm, 1), row),
            pl.BlockSpec((bm, H), row),
            pl.BlockSpec(w1a.shape, fix),
            pl.BlockSpec((1, H), fix),
            pl.BlockSpec(w1b.shape, fix),
            pl.BlockSpec((1, H), fix),
            pl.BlockSpec(wu1.shape, fix),
            pl.BlockSpec((1, 2 * H), fix),
            pl.BlockSpec(wu2.shape, fix),
            pl.BlockSpec((1, H), fix),
        ],
        out_specs=pl.BlockSpec((bm, H), row),
        out_shape=jax.ShapeDtypeStruct((NPAD, H), jnp.float32),
    )(p0, p1, p2, p3, cnt, ns, w1a, b1a.reshape(1, -1), w1b,
      b1b.reshape(1, -1), wu1, bu1.reshape(1, -1), wu2, bu2.reshape(1, -1))


def kernel(node_states, rel0, rel1, W0a, b0a, W0b, b0b, W1a, b1a, W1b, b1b,
           Wu1, bu1, Wu2, bu2):
    r0 = rel0.astype(jnp.int32).reshape(-1, 2 * CH)
    idxa = r0[:, 0::2]
    idxb = r0[:, 1::2]
    idx1 = rel1.astype(jnp.int32).reshape(-1, CH)
    lane = jnp.arange(CH, dtype=jnp.int32)
    spad = N + lane
    idxa_p = _pad_chunks(idxa, NW * 8, spad)
    idxb_p = _pad_chunks(idxb, NW * 8, spad)
    c1h = idx1.shape[0] // 2
    idx1_h = [_pad_chunks(idx1[:c1h], NW * 8, spad),
              _pad_chunks(idx1[c1h:], NW * 8, spad)]
    table = jnp.concatenate(
        [node_states, jnp.zeros((NPAD - N, H), jnp.float32)])

    hc = idxa_p.shape[0] // 2
    zeros = jnp.zeros((NPAD, H), jnp.float32)
    zcnt = jnp.zeros((NPAD,), jnp.int32)

    halves = []
    for k in range(2):
        ia = idxa_p[k * hc:(k + 1) * hc]
        ib = idxb_p[k * hc:(k + 1) * hc]
        i1 = idx1_h[k]
        inp = _sc_gather(table, ia, ib)
        out = _tc_mlp(inp, W0a, b0a, W0b, b0b, bm=4096)
        halves.append(_sc_scatter(out, ia, ib, i1, zeros, zcnt))

    (pA, cA), (pB, cB) = halves
    cnt = ((cA[0] + cA[1]) + (cB[0] + cB[1])).astype(jnp.float32)
    out = _tc_update(pA[0], pA[1], pB[0], pB[1], cnt.reshape(NPAD, 1), table,
                     W1a, b1a, W1b, b1b, Wu1, bu1, Wu2, bu2)
    return out[:N]


# speedup vs baseline: 1.1977x; 1.1463x over previous
import jax
import jax.numpy as jnp
from jax import lax
from jax.experimental import pallas as pl
from jax.experimental.pallas import tpu as pltpu
from jax.experimental.pallas import tpu_sc as plsc

N = 10000
NPAD = 10240
H = 128
CH = 128
NW = 32
NS = 16

_mesh = plsc.VectorSubcoreMesh(core_axis_name="c", subcore_axis_name="s")


def _pad_chunks(idx, mult, pad_row):
    pad = (-idx.shape[0]) % mult
    if pad:
        idx = jnp.concatenate([idx, jnp.broadcast_to(pad_row, (pad, CH))])
    return idx


def _sc_gather(table, idxa, idxb):
    nloc = idxa.shape[0] // NW
    HF = CH // 2

    def body(table_hbm, idxa_hbm, idxb_hbm, out_hbm,
             idxva, idxvb, buf, tbl, s0, s1):
        s = lax.axis_index("s")
        w = s * 2 + lax.axis_index("c")
        slab = NPAD // NS
        pltpu.sync_copy(table_hbm.at[pl.ds(s * slab, slab)],
                        tbl.at[pl.ds(s * slab, slab)])
        pltpu.sync_copy(idxa_hbm.at[pl.ds(w * nloc * CH, nloc * CH)], idxva)
        pltpu.sync_copy(idxb_hbm.at[pl.ds(w * nloc * CH, nloc * CH)], idxvb)
        plsc.subcore_barrier()
        sems = (s0, s1)
        T = 4 * nloc
        pend = [None, None]
        for t in range(T + 1):
            p = t & 1
            if t < T:
                j, ab, half = t >> 2, (t >> 1) & 1, t & 1
                iv = idxva if ab == 0 else idxvb
                d = pltpu.async_copy(
                    tbl.at[iv.at[pl.ds(j * CH + half * HF, HF)]],
                    buf.at[p], sems[p])
            if t >= 1:
                t1 = t - 1
                q = t1 & 1
                j1, ab1, half1 = t1 >> 2, (t1 >> 1) & 1, t1 & 1
                chunk = w * nloc + j1
                pend[q].wait()
                pltpu.sync_copy(
                    buf.at[q],
                    out_hbm.at[pl.ds(chunk * CH + half1 * HF, HF),
                               pl.ds(ab1 * H, H)])
            if t < T:
                pend[p] = d

    f = pl.kernel(
        body,
        out_type=jax.ShapeDtypeStruct((idxa.shape[0] * CH, 2 * H),
                                      jnp.float32),
        mesh=_mesh,
        scratch_types=[pltpu.VMEM((nloc * CH,), jnp.int32),
                       pltpu.VMEM((nloc * CH,), jnp.int32),
                       pltpu.VMEM((2, HF, H), jnp.float32),
                       pltpu.VMEM_SHARED((NPAD, H), jnp.float32),
                       pltpu.SemaphoreType.DMA,
                       pltpu.SemaphoreType.DMA],
    )
    return f(table, idxa.reshape(-1), idxb.reshape(-1))


def _sc_scatter(rows0, idxa, idxb, idx1, zeros_hbm, zc_hbm):
    n0 = idxa.shape[0] // NW
    n1 = idx1.shape[0] // NW
    slab = NPAD // NS

    def body(rows0_hbm, idxa_hbm, idxb_hbm, idx1_hbm, z_hbm, zc, out_hbm,
             cnt_hbm, ga, gb, idxv1, buf, onesv, acc, acc_cnt, s0, s1):
        c = lax.axis_index("c")
        s = lax.axis_index("s")
        w = s * 2 + c
        pltpu.sync_copy(z_hbm.at[pl.ds(s * slab, slab)],
                        acc.at[pl.ds(s * slab, slab)])
        pltpu.sync_copy(zc.at[pl.ds(s * slab, slab)],
                        acc_cnt.at[pl.ds(s * slab, slab)])
        ones = jnp.ones((16,), jnp.int32)
        for k in range(CH // 16):
            onesv[pl.ds(k * 16, 16)] = ones
        plsc.subcore_barrier()

        pltpu.sync_copy(idx1_hbm.at[pl.ds(w * n1, n1)], idxv1)

        sems = (s0, s1)
        T = 2 * n0
        pend = [None, None]
        for t in range(T + 1):
            p = t & 1
            if t < T:
                j, st = t >> 1, t & 1
                if st == 0 and j % 8 == 0:
                    g = j // 8
                    gp = g & 1
                    pltpu.sync_copy(
                        idxa_hbm.at[pl.ds(w * n0 + g * 8, 8)], ga.at[gp])
                    pltpu.sync_copy(
                        idxb_hbm.at[pl.ds(w * n0 + g * 8, 8)], gb.at[gp])
                chunk = w * n0 + j
                cols = pl.ds(0, H) if st == 0 else pl.ds(H, H)
                r = pltpu.async_copy(rows0_hbm.at[pl.ds(chunk * CH, CH), cols],
                                     buf.at[p], sems[p])
            if t >= 1:
                q = (t - 1) & 1
                j1, st1 = (t - 1) >> 1, (t - 1) & 1
                gp1 = (j1 // 8) & 1
                idxref = (ga if st1 == 0 else gb).at[gp1, j1 % 8]
                pend[q].wait()
                pltpu.sync_copy(buf.at[q], acc.at[idxref], add=True)
            if t < T:
                pend[p] = r

        def it1(j, carry):
            pltpu.sync_copy(onesv, acc_cnt.at[idxv1.at[j]], add=True)
            return carry

        lax.fori_loop(0, n1, it1, 0)

        plsc.subcore_barrier()
        pltpu.sync_copy(acc.at[pl.ds(s * slab, slab)],
                        out_hbm.at[c, pl.ds(s * slab, slab)])
        pltpu.sync_copy(acc_cnt.at[pl.ds(s * slab, slab)],
                        cnt_hbm.at[c, pl.ds(s * slab, slab)])

    f = pl.kernel(
        body,
        out_type=(jax.ShapeDtypeStruct((2, NPAD, H), jnp.float32),
                  jax.ShapeDtypeStruct((2, NPAD), jnp.int32)),
        mesh=_mesh,
        scratch_types=[pltpu.VMEM((2, 8, CH), jnp.int32),
                       pltpu.VMEM((2, 8, CH), jnp.int32),
                       pltpu.VMEM((n1, CH), jnp.int32),
                       pltpu.VMEM((2, CH, H), jnp.float32),
                       pltpu.VMEM((CH,), jnp.int32),
                       pltpu.VMEM_SHARED((NPAD, H), jnp.float32),
                       pltpu.VMEM_SHARED((NPAD,), jnp.int32),
                       pltpu.SemaphoreType.DMA,
                       pltpu.SemaphoreType.DMA],
    )
    return f(rows0, idxa, idxb, idx1, zeros_hbm, zc_hbm)


def _mlp_body(x_ref, wa_ref, ba_ref, wb_ref, bb_ref, o_ref):
    x = x_ref[...].astype(jnp.bfloat16)
    wa = wa_ref[...].astype(jnp.bfloat16)
    h = lax.dot_general(x, wa, (((1,), (1,)), ((), ())),
                        preferred_element_type=jnp.float32)
    h = jnp.maximum(h + ba_ref[...], 0.0).astype(jnp.bfloat16)
    wb = wb_ref[...].astype(jnp.bfloat16)
    o = lax.dot_general(h, wb, (((1,), (1,)), ((), ())),
                        preferred_element_type=jnp.float32)
    o_ref[...] = o + bb_ref[...]


def _tc_mlp(x, wa, ba, wb, bb, bm):
    m, k = x.shape
    ko = wb.shape[0]
    return pl.pallas_call(
        _mlp_body,
        grid=(m // bm,),
        in_specs=[
            pl.BlockSpec((bm, k), lambda i: (i, 0)),
            pl.BlockSpec(wa.shape, lambda i: (0, 0)),
            pl.BlockSpec((1, ba.shape[0]), lambda i: (0, 0)),
            pl.BlockSpec(wb.shape, lambda i: (0, 0)),
            pl.BlockSpec((1, bb.shape[0]), lambda i: (0, 0)),
        ],
        out_specs=pl.BlockSpec((bm, ko), lambda i: (i, 0)),
        out_shape=jax.ShapeDtypeStruct((m, ko), jnp.float32),
    )(x, wa, ba.reshape(1, -1), wb, bb.reshape(1, -1))


def _update_body(p0_ref, p1_ref, p2_ref, p3_ref, cnt_ref, ns_ref, w1a_ref,
                 b1a_ref, w1b_ref, b1b_ref, wu1_ref, bu1_ref, wu2_ref,
                 bu2_ref, o_ref):
    ns = ns_ref[...]
    h1 = lax.dot_general(ns, w1a_ref[...], (((1,), (1,)), ((), ())),
                         preferred_element_type=jnp.float32)
    h1 = jnp.maximum(h1 + b1a_ref[...], 0.0)
    m1 = lax.dot_general(h1, w1b_ref[...], (((1,), (1,)), ((), ())),
                         preferred_element_type=jnp.float32)
    m1 = m1 + b1b_ref[...]
    sm = ((p0_ref[...] + p1_ref[...]) + (p2_ref[...] + p3_ref[...])
          + cnt_ref[...] * m1)
    wu1 = wu1_ref[...]
    h = lax.dot_general(sm, wu1[:, :H], (((1,), (1,)), ((), ())),
                        preferred_element_type=jnp.float32)
    h = h + lax.dot_general(ns, wu1[:, H:], (((1,), (1,)), ((), ())),
                            preferred_element_type=jnp.float32)
    h = jnp.maximum(h + bu1_ref[...], 0.0)
    o = lax.dot_general(h, wu2_ref[...], (((1,), (1,)), ((), ())),
                        preferred_element_type=jnp.float32)
    o_ref[...] = o + bu2_ref[...]


def _tc_update(p0, p1, p2, p3, cnt, ns, w1a, b1a, w1b, b1b, wu1, bu1, wu2,
               bu2):
    bm = 1280
    row = lambda i: (i, 0)
    fix = lambda i: (0, 0)
    return pl.pallas_call(
        _update_body,
        grid=(NPAD // bm,),
        in_specs=[
            pl.BlockSpec((bm, H), row),
            pl.BlockSpec((bm, H), row),
            pl.BlockSpec((bm, H), row),
            pl.BlockSpec((bm, H), row),
            pl.BlockSpec((bm, 1), row),
            pl.BlockSpec((bm, H), row),
            pl.BlockSpec(w1a.shape, fix),
            pl.BlockSpec((1, H), fix),
            pl.BlockSpec(w1b.shape, fix),
            pl.BlockSpec((1, H), fix),
            pl.BlockSpec(wu1.shape, fix),
            pl.BlockSpec((1, 2 * H), fix),
            pl.BlockSpec(wu2.shape, fix),
            pl.BlockSpec((1, H), fix),
        ],
        out_specs=pl.BlockSpec((bm, H), row),
        out_shape=jax.ShapeDtypeStruct((NPAD, H), jnp.float32),
    )(p0, p1, p2, p3, cnt, ns, w1a, b1a.reshape(1, -1), w1b,
      b1b.reshape(1, -1), wu1, bu1.reshape(1, -1), wu2, bu2.reshape(1, -1))


def kernel(node_states, rel0, rel1, W0a, b0a, W0b, b0b, W1a, b1a, W1b, b1b,
           Wu1, bu1, Wu2, bu2):
    r0 = rel0.astype(jnp.int32).reshape(-1, 2 * CH)
    idxa = r0[:, 0::2]
    idxb = r0[:, 1::2]
    idx1 = rel1.astype(jnp.int32).reshape(-1, CH)
    lane = jnp.arange(CH, dtype=jnp.int32)
    spad = N + lane
    idxa_p = _pad_chunks(idxa, NW * 8, spad)
    idxb_p = _pad_chunks(idxb, NW * 8, spad)
    c1h = idx1.shape[0] // 2
    idx1_h = [_pad_chunks(idx1[:c1h], NW * 8, spad),
              _pad_chunks(idx1[c1h:], NW * 8, spad)]
    table = jnp.concatenate(
        [node_states, jnp.zeros((NPAD - N, H), jnp.float32)])

    hc = idxa_p.shape[0] // 2
    zeros = jnp.zeros((NPAD, H), jnp.float32)
    zcnt = jnp.zeros((NPAD,), jnp.int32)

    halves = []
    for k in range(2):
        ia = idxa_p[k * hc:(k + 1) * hc]
        ib = idxb_p[k * hc:(k + 1) * hc]
        i1 = idx1_h[k]
        inp = _sc_gather(table, ia, ib)
        out = _tc_mlp(inp, W0a, b0a, W0b, b0b, bm=4096)
        halves.append(_sc_scatter(out, ia, ib, i1, zeros, zcnt))

    (pA, cA), (pB, cB) = halves
    cnt = ((cA[0] + cA[1]) + (cB[0] + cB[1])).astype(jnp.float32)
    out = _tc_update(pA[0], pA[1], pB[0], pB[1], cnt.reshape(NPAD, 1), table,
                     W1a, b1a, W1b, b1b, Wu1, bu1, Wu2, bu2)
    return out[:N]
